# Initial kernel scaffold; baseline (speedup 1.0000x reference)
import jax
import jax.numpy as jnp
from jax.experimental import pallas as pl


def kernel(x, embed_real, embed_imag, gamma, beta):
    raise NotImplementedError("write your pallas kernel here")


# speedup vs baseline: 1.2498x; 1.2498x over previous
import functools

import jax
import jax.numpy as jnp
from jax import lax
from jax.experimental import pallas as pl
from jax.experimental.pallas import tpu as pltpu
from jax.experimental.pallas import tpu_sc as plsc

NC = 2
NS = 16
L = 16
CHUNK = 64


def _lane_sum(x):
    lanes = lax.iota(jnp.int32, L)
    for sh in (1, 2, 4, 8):
        perm = lanes ^ sh
        x = x + x.at[perm].get(mode="promise_in_bounds")
    return x


def _rsqrt_nr(x, iters):
    i = lax.bitcast_convert_type(x, jnp.int32)
    i = 0x5F3759DF - (i >> 1)
    y = lax.bitcast_convert_type(i, jnp.float32)
    xh = x * 0.5
    for _ in range(iters):
        y = y * (1.5 - xh * y * y)
    return y


def _make_sc_kernel(n_rows, dim):
    assert dim == 128 and n_rows % (NC * NS * CHUNK) == 0
    rows_per_tile = n_rows // (NC * NS)
    n_chunks = rows_per_tile // CHUNK
    nj = dim // L

    mesh = plsc.VectorSubcoreMesh(
        core_axis_name="c", subcore_axis_name="s",
        num_cores=NC, num_subcores=NS)

    @functools.partial(
        pl.kernel,
        out_type=(
            jax.ShapeDtypeStruct((n_rows, dim), jnp.float32),
            jax.ShapeDtypeStruct((n_rows, dim), jnp.float32),
        ),
        mesh=mesh,
        scratch_types=[
            pltpu.VMEM((CHUNK,), jnp.int32),
            pltpu.VMEM((CHUNK, dim), jnp.float32),
            pltpu.VMEM((CHUNK, dim), jnp.float32),
            pltpu.VMEM((CHUNK, dim), jnp.float32),
            pltpu.VMEM((CHUNK, dim), jnp.float32),
            pltpu.VMEM((dim,), jnp.float32),
            pltpu.VMEM((dim,), jnp.float32),
            pltpu.SemaphoreType.DMA,
            pltpu.SemaphoreType.DMA,
        ],
    )
    def sc_kernel(x_hbm, er_hbm, ei_hbm, g_hbm, b_hbm, outr_hbm, outi_hbm,
                  idx_v, re_v, im_v, or_v, oi_v, g_v, b_v, sem_r, sem_i):
        wid = lax.axis_index("s") * NC + lax.axis_index("c")
        tile_base = wid * rows_per_tile

        pltpu.sync_copy(g_hbm, g_v)
        pltpu.sync_copy(b_hbm, b_v)
        gs = [g_v[pl.ds(L * j, L)] for j in range(nj)]
        bs = [b_v[pl.ds(L * j, L)] for j in range(nj)]

        def row_body(r, _):
            mags, ys = [], []
            acc = jnp.zeros((L,), jnp.float32)
            acc2 = jnp.zeros((L,), jnp.float32)
            for j in range(nj):
                re = re_v[r, pl.ds(L * j, L)]
                im = im_v[r, pl.ds(L * j, L)]
                msq = re * re + im * im
                y = _rsqrt_nr(msq, 2)
                mag = msq * y
                acc = acc + mag
                acc2 = acc2 + msq
                mags.append(mag)
                ys.append(y)
            mu_v = _lane_sum(acc) * (1.0 / dim)
            ex2 = _lane_sum(acc2) * (1.0 / dim)
            var = ex2 - mu_v * mu_v
            rs = _rsqrt_nr(var + 1e-5, 3)
            for j in range(nj):
                re = re_v[r, pl.ds(L * j, L)]
                im = im_v[r, pl.ds(L * j, L)]
                normed = (mags[j] - mu_v) * (rs * gs[j]) + bs[j]
                s = normed * ys[j]
                or_v[r, pl.ds(L * j, L)] = re * s
                oi_v[r, pl.ds(L * j, L)] = im * s
            return _

        def chunk_body(k, _):
            base = tile_base + k * CHUNK
            pltpu.sync_copy(x_hbm.at[pl.ds(base, CHUNK)], idx_v)
            cr = pltpu.async_copy(er_hbm.at[idx_v], re_v, sem_r)
            ci = pltpu.async_copy(ei_hbm.at[idx_v], im_v, sem_i)
            cr.wait()
            ci.wait()
            lax.fori_loop(0, CHUNK, row_body, None)
            pltpu.sync_copy(or_v, outr_hbm.at[pl.ds(base, CHUNK)])
            pltpu.sync_copy(oi_v, outi_hbm.at[pl.ds(base, CHUNK)])
            return _

        lax.fori_loop(0, n_chunks, chunk_body, None)

    return sc_kernel


def kernel(x, embed_real, embed_imag, gamma, beta):
    b, h = x.shape
    v, d = embed_real.shape
    xf = x.reshape(b * h)
    sc = _make_sc_kernel(b * h, d)
    outr, outi = sc(xf, embed_real, embed_imag, gamma, beta)
    return lax.complex(outr, outi).reshape(b, h, d)
